# Initial kernel scaffold; baseline (speedup 1.0000x reference)
import functools

import jax
import jax.numpy as jnp
from jax import lax
from jax.experimental import pallas as pl
from jax.experimental.pallas import tpu as pltpu
from jax.experimental.pallas import tpu_sc as plsc

N = 10000
E = 320000
D = 128
A = 5

NC = 2
NS = 16
NW = NC * NS
EPW = E // NW
CH = 80
NCHUNK = EPW // CH
RPT = N // NS

_mesh = plsc.VectorSubcoreMesh(core_axis_name="c", subcore_axis_name="s")


@functools.partial(
    pl.kernel,
    out_type=jax.ShapeDtypeStruct((NC * N, D), jnp.float32),
    mesh=_mesh,
    scratch_types=[
        pltpu.VMEM((CH,), jnp.int32),
        pltpu.VMEM((CH,), jnp.int32),
        pltpu.VMEM((CH,), jnp.int32),
        pltpu.VMEM((CH, D), jnp.float32),
        pltpu.VMEM((CH, D), jnp.float32),
        pltpu.VMEM((CH, D), jnp.float32),
        pltpu.MemorySpace.VMEM_SHARED((N, D), jnp.float32),
        pltpu.SemaphoreType.DMA,
        pltpu.SemaphoreType.DMA,
    ],
)
def _sc_conv(src_hbm, dst_hbm, typ_hbm, x_hbm, rel_hbm, out_hbm,
             src_v, dst_v, typ_v, xrows, relrows, msg, agg, sem1, sem2):
    c = lax.axis_index("c")
    s = lax.axis_index("s")
    wid = c * NS + s

    def _zrow(r, _):
        for j in range(D // 16):
            msg[r, pl.ds(j * 16, 16)] = jnp.zeros((16,), jnp.float32)
        return 0
    lax.fori_loop(0, CH, _zrow, 0)

    base = s * RPT
    for t in range(7):
        pltpu.sync_copy(msg, agg.at[pl.ds(base + t * CH, CH)])
    pltpu.sync_copy(msg.at[pl.ds(0, 65)], agg.at[pl.ds(base + 560, 65)])
    plsc.subcore_barrier()

    ebase = wid * EPW

    def _chunk(k, _):
        off = ebase + k * CH
        pltpu.sync_copy(src_hbm.at[pl.ds(off, CH)], src_v)
        pltpu.sync_copy(typ_hbm.at[pl.ds(off, CH)], typ_v)
        pltpu.sync_copy(dst_hbm.at[pl.ds(off, CH)], dst_v)
        g1 = pltpu.async_copy(x_hbm.at[src_v], xrows, sem1)
        g2 = pltpu.async_copy(rel_hbm.at[typ_v], relrows, sem2)
        g1.wait()
        g2.wait()

        def _rot(r, _):
            for j in range(D // 32):
                hre = xrows[r, pl.ds(j * 16, 16)]
                him = xrows[r, pl.ds(64 + j * 16, 16)]
                rre = relrows[r, pl.ds(j * 16, 16)]
                rim = relrows[r, pl.ds(64 + j * 16, 16)]
                msg[r, pl.ds(j * 16, 16)] = hre * rre - him * rim
                msg[r, pl.ds(64 + j * 16, 16)] = hre * rim + him * rre
            return 0
        lax.fori_loop(0, CH, _rot, 0)

        pltpu.sync_copy(msg, agg.at[dst_v], add=True)
        return 0

    lax.fori_loop(0, NCHUNK, _chunk, 0)
    plsc.subcore_barrier()

    pltpu.sync_copy(agg.at[pl.ds(base, RPT)], out_hbm.at[pl.ds(c * N + base, RPT)])


_BLK = 400


def _tc_conv_body(p0_ref, p1_ref, w_ref, b_ref, o_ref):
    acc = p0_ref[...] + p1_ref[...]
    o_ref[...] = jnp.maximum(acc @ w_ref[...] + b_ref[...], 0.0)


_tc_conv = pl.pallas_call(
    _tc_conv_body,
    grid=(N // _BLK,),
    in_specs=[
        pl.BlockSpec((_BLK, D), lambda i: (i, 0)),
        pl.BlockSpec((_BLK, D), lambda i: (i, 0)),
        pl.BlockSpec((D, D), lambda i: (0, 0)),
        pl.BlockSpec((1, D), lambda i: (0, 0)),
    ],
    out_specs=pl.BlockSpec((_BLK, D), lambda i: (i, 0)),
    out_shape=jax.ShapeDtypeStruct((N, D), jnp.float32),
)


def _tc_head_body(p0_ref, p1_ref, w2_ref, b2_ref, wm1_ref, bm1_ref,
                  wm2_ref, bm2_ref, wav_ref, bav_ref, o_ref):
    x = jnp.maximum((p0_ref[...] + p1_ref[...]) @ w2_ref[...] + b2_ref[...], 0.0)
    h = jnp.maximum(x @ wm1_ref[...] + bm1_ref[...], 0.0)
    h = jnp.maximum(h @ wm2_ref[...] + bm2_ref[...], 0.0)
    av = h @ wav_ref[...] + bav_ref[...]
    adv = av[:, :A]
    val = av[:, A:A + 1]
    o_ref[...] = val + adv - jnp.mean(adv, axis=-1, keepdims=True)


_H = 128

_tc_head = pl.pallas_call(
    _tc_head_body,
    grid=(N // _BLK,),
    in_specs=[
        pl.BlockSpec((_BLK, D), lambda i: (i, 0)),
        pl.BlockSpec((_BLK, D), lambda i: (i, 0)),
        pl.BlockSpec((D, D), lambda i: (0, 0)),
        pl.BlockSpec((1, D), lambda i: (0, 0)),
        pl.BlockSpec((D, _H), lambda i: (0, 0)),
        pl.BlockSpec((1, _H), lambda i: (0, 0)),
        pl.BlockSpec((_H, _H), lambda i: (0, 0)),
        pl.BlockSpec((1, _H), lambda i: (0, 0)),
        pl.BlockSpec((_H, A + 1), lambda i: (0, 0)),
        pl.BlockSpec((1, A + 1), lambda i: (0, 0)),
    ],
    out_specs=pl.BlockSpec((_BLK, A), lambda i: (i, 0)),
    out_shape=jax.ShapeDtypeStruct((N, A), jnp.float32),
)


def kernel(edge_index, edge_type, entity_emb, rel_emb, W1, b1, W2, b2,
           Wm1, bm1, Wm2, bm2, Wa, ba, Wv, bv):
    src = edge_index[0].astype(jnp.int32)
    dst = edge_index[1].astype(jnp.int32)
    typ = edge_type.astype(jnp.int32)

    p = _sc_conv(src, dst, typ, entity_emb, rel_emb)
    x1 = _tc_conv(p[:N], p[N:], W1, b1.reshape(1, D))
    p2 = _sc_conv(src, dst, typ, x1, rel_emb)

    wav = jnp.concatenate([Wa, Wv], axis=1)
    bav = jnp.concatenate([ba, bv]).reshape(1, A + 1)
    return _tc_head(p2[:N], p2[N:], W2, b2.reshape(1, D),
                    Wm1, bm1.reshape(1, _H), Wm2, bm2.reshape(1, _H),
                    wav, bav)


# speedup vs baseline: 2.6874x; 2.6874x over previous
import functools

import jax
import jax.numpy as jnp
from jax import lax
from jax.experimental import pallas as pl
from jax.experimental.pallas import tpu as pltpu
from jax.experimental.pallas import tpu_sc as plsc

N = 10000
E = 320000
D = 128
A = 5

NC = 2
NS = 16
NW = NC * NS
EPW = E // NW
CH = 80
NCHUNK = EPW // CH
RPT = 624

_mesh = plsc.VectorSubcoreMesh(core_axis_name="c", subcore_axis_name="s")


@functools.partial(
    pl.kernel,
    out_type=jax.ShapeDtypeStruct((NC * N, D), jnp.float32),
    mesh=_mesh,
    scratch_types=[
        pltpu.VMEM((CH,), jnp.int32),
        pltpu.VMEM((CH,), jnp.int32),
        pltpu.VMEM((CH,), jnp.int32),
        pltpu.VMEM((CH, D), jnp.float32),
        pltpu.VMEM((CH, D), jnp.float32),
        pltpu.VMEM((CH, D), jnp.float32),
        pltpu.MemorySpace.VMEM_SHARED((N, D), jnp.float32),
        pltpu.SemaphoreType.DMA,
        pltpu.SemaphoreType.DMA,
    ],
)
def _sc_conv(src_hbm, dst_hbm, typ_hbm, x_hbm, rel_hbm, out_hbm,
             src_v, dst_v, typ_v, xrows, relrows, msg, agg, sem1, sem2):
    c = lax.axis_index("c")
    s = lax.axis_index("s")
    wid = c * NS + s

    def _zrow(r, _):
        for j in range(D // 16):
            msg[r, pl.ds(j * 16, 16)] = jnp.zeros((16,), jnp.float32)
        return 0
    lax.fori_loop(0, CH, _zrow, 0)

    base = s * RPT
    for t in range(7):
        pltpu.sync_copy(msg, agg.at[pl.ds(base + t * CH, CH)])
    pltpu.sync_copy(msg.at[pl.ds(0, 64)], agg.at[pl.ds(base + 560, 64)])

    @pl.when(s == NS - 1)
    def _zero_tail():
        pltpu.sync_copy(msg.at[pl.ds(0, 16)], agg.at[pl.ds(NS * RPT, 16)])

    plsc.subcore_barrier()

    ebase = wid * EPW

    def _chunk(k, _):
        off = ebase + k * CH
        pltpu.sync_copy(src_hbm.at[pl.ds(off, CH)], src_v)
        pltpu.sync_copy(typ_hbm.at[pl.ds(off, CH)], typ_v)
        pltpu.sync_copy(dst_hbm.at[pl.ds(off, CH)], dst_v)
        g1 = pltpu.async_copy(x_hbm.at[src_v], xrows, sem1)
        g2 = pltpu.async_copy(rel_hbm.at[typ_v], relrows, sem2)
        g1.wait()
        g2.wait()

        def _rot(r, _):
            for j in range(D // 32):
                hre = xrows[r, pl.ds(j * 16, 16)]
                him = xrows[r, pl.ds(64 + j * 16, 16)]
                rre = relrows[r, pl.ds(j * 16, 16)]
                rim = relrows[r, pl.ds(64 + j * 16, 16)]
                msg[r, pl.ds(j * 16, 16)] = hre * rre - him * rim
                msg[r, pl.ds(64 + j * 16, 16)] = hre * rim + him * rre
            return 0
        lax.fori_loop(0, CH, _rot, 0)

        pltpu.sync_copy(msg, agg.at[dst_v], add=True)
        return 0

    lax.fori_loop(0, NCHUNK, _chunk, 0)
    plsc.subcore_barrier()

    pltpu.sync_copy(agg.at[pl.ds(base, RPT)], out_hbm.at[pl.ds(c * N + base, RPT)])

    @pl.when(s == NS - 1)
    def _write_tail():
        pltpu.sync_copy(agg.at[pl.ds(NS * RPT, 16)],
                        out_hbm.at[pl.ds(c * N + NS * RPT, 16)])


_BLK = 400


def _tc_conv_body(p0_ref, p1_ref, w_ref, b_ref, o_ref):
    acc = p0_ref[...] + p1_ref[...]
    o_ref[...] = jnp.maximum(acc @ w_ref[...] + b_ref[...], 0.0)


_tc_conv = pl.pallas_call(
    _tc_conv_body,
    grid=(N // _BLK,),
    in_specs=[
        pl.BlockSpec((_BLK, D), lambda i: (i, 0)),
        pl.BlockSpec((_BLK, D), lambda i: (i, 0)),
        pl.BlockSpec((D, D), lambda i: (0, 0)),
        pl.BlockSpec((1, D), lambda i: (0, 0)),
    ],
    out_specs=pl.BlockSpec((_BLK, D), lambda i: (i, 0)),
    out_shape=jax.ShapeDtypeStruct((N, D), jnp.float32),
)


def _tc_head_body(p0_ref, p1_ref, w2_ref, b2_ref, wm1_ref, bm1_ref,
                  wm2_ref, bm2_ref, wav_ref, bav_ref, o_ref):
    x = jnp.maximum((p0_ref[...] + p1_ref[...]) @ w2_ref[...] + b2_ref[...], 0.0)
    h = jnp.maximum(x @ wm1_ref[...] + bm1_ref[...], 0.0)
    h = jnp.maximum(h @ wm2_ref[...] + bm2_ref[...], 0.0)
    av = h @ wav_ref[...] + bav_ref[...]
    adv = av[:, :A]
    val = av[:, A:A + 1]
    o_ref[...] = val + adv - jnp.mean(adv, axis=-1, keepdims=True)


_H = 128

_tc_head = pl.pallas_call(
    _tc_head_body,
    grid=(N // _BLK,),
    in_specs=[
        pl.BlockSpec((_BLK, D), lambda i: (i, 0)),
        pl.BlockSpec((_BLK, D), lambda i: (i, 0)),
        pl.BlockSpec((D, D), lambda i: (0, 0)),
        pl.BlockSpec((1, D), lambda i: (0, 0)),
        pl.BlockSpec((D, _H), lambda i: (0, 0)),
        pl.BlockSpec((1, _H), lambda i: (0, 0)),
        pl.BlockSpec((_H, _H), lambda i: (0, 0)),
        pl.BlockSpec((1, _H), lambda i: (0, 0)),
        pl.BlockSpec((_H, A + 1), lambda i: (0, 0)),
        pl.BlockSpec((1, A + 1), lambda i: (0, 0)),
    ],
    out_specs=pl.BlockSpec((_BLK, A), lambda i: (i, 0)),
    out_shape=jax.ShapeDtypeStruct((N, A), jnp.float32),
)


def kernel(edge_index, edge_type, entity_emb, rel_emb, W1, b1, W2, b2,
           Wm1, bm1, Wm2, bm2, Wa, ba, Wv, bv):
    src = edge_index[0].astype(jnp.int32)
    dst = edge_index[1].astype(jnp.int32)
    typ = edge_type.astype(jnp.int32)

    p = _sc_conv(src, dst, typ, entity_emb, rel_emb)
    x1 = _tc_conv(p[:N], p[N:], W1, b1.reshape(1, D))
    p2 = _sc_conv(src, dst, typ, x1, rel_emb)

    wav = jnp.concatenate([Wa, Wv], axis=1)
    bav = jnp.concatenate([ba, bv]).reshape(1, A + 1)
    return _tc_head(p2[:N], p2[N:], W2, b2.reshape(1, D),
                    Wm1, bm1.reshape(1, _H), Wm2, bm2.reshape(1, _H),
                    wav, bav)


# speedup vs baseline: 3.1464x; 1.1708x over previous
import functools

import jax
import jax.numpy as jnp
from jax import lax
from jax.experimental import pallas as pl
from jax.experimental.pallas import tpu as pltpu
from jax.experimental.pallas import tpu_sc as plsc

N = 10000
E = 320000
D = 128
R = 32
A = 5

NC = 2
NS = 16
NW = NC * NS
EPW = E // NW
CH = 80
NCHUNK = EPW // CH
RPT = 624

_mesh = plsc.VectorSubcoreMesh(core_axis_name="c", subcore_axis_name="s")


@functools.partial(
    pl.kernel,
    out_type=jax.ShapeDtypeStruct((NC * N, D), jnp.float32),
    mesh=_mesh,
    scratch_types=[
        pltpu.VMEM((EPW,), jnp.int32),
        pltpu.VMEM((NCHUNK, CH), jnp.int32),
        [pltpu.VMEM((CH, D), jnp.float32)] * 2,
        pltpu.MemorySpace.VMEM_SHARED((N, D), jnp.float32),
        [pltpu.SemaphoreType.DMA] * 2,
    ],
)
def _sc_conv(cidx_hbm, dst_hbm, t_hbm, out_hbm,
             cidx_v, dst_v, rows, agg, semx):
    c = lax.axis_index("c")
    s = lax.axis_index("s")
    wid = c * NS + s

    pltpu.sync_copy(cidx_hbm.at[wid], cidx_v)
    pltpu.sync_copy(dst_hbm.at[wid], dst_v)

    def _zrow(r, _):
        for j in range(D // 16):
            rows[0][r, pl.ds(j * 16, 16)] = jnp.zeros((16,), jnp.float32)
        return 0
    lax.fori_loop(0, CH, _zrow, 0)

    base = s * RPT
    for t in range(RPT // CH):
        pltpu.sync_copy(rows[0], agg.at[pl.ds(base + t * CH, CH)])
    if RPT % CH:
        pltpu.sync_copy(rows[0].at[pl.ds(0, RPT % CH)],
                        agg.at[pl.ds(base + (RPT // CH) * CH, RPT % CH)])

    @pl.when(s == NS - 1)
    def _zero_tail():
        pltpu.sync_copy(rows[0].at[pl.ds(0, 16)], agg.at[pl.ds(NS * RPT, 16)])

    plsc.subcore_barrier()

    def _fetch(k, b):
        pltpu.async_copy(t_hbm.at[cidx_v.at[pl.ds(k * CH, CH)]], rows[b], semx[b])

    def _consume(k, b):
        pltpu.make_async_copy(t_hbm.at[cidx_v.at[pl.ds(k * CH, CH)]],
                              rows[b], semx[b]).wait()
        pltpu.sync_copy(rows[b], agg.at[dst_v.at[k]], add=True)

    _fetch(0, 0)

    def _pair(i, _):
        _fetch(2 * i + 1, 1)
        _consume(2 * i, 0)
        _fetch(2 * i + 2, 0)
        _consume(2 * i + 1, 1)
        return 0

    lax.fori_loop(0, (NCHUNK - 1) // 2, _pair, 0)
    _consume(NCHUNK - 1, 0)
    plsc.subcore_barrier()

    pltpu.sync_copy(agg.at[pl.ds(base, RPT)], out_hbm.at[pl.ds(c * N + base, RPT)])

    @pl.when(s == NS - 1)
    def _write_tail():
        pltpu.sync_copy(agg.at[pl.ds(NS * RPT, 16)],
                        out_hbm.at[pl.ds(c * N + NS * RPT, 16)])


_BLK = 400
_DH = D // 2


def _tc_rot_body(x_ref, rel_ref, o_ref):
    t = pl.program_id(0)
    onehot = (lax.broadcasted_iota(jnp.int32, (R, 1), 0) == t).astype(jnp.float32)
    relrow = jnp.sum(rel_ref[...] * onehot, axis=0, keepdims=True)
    hre = x_ref[:, :_DH]
    him = x_ref[:, _DH:]
    rre = relrow[:, :_DH]
    rim = relrow[:, _DH:]
    o_ref[:, :_DH] = hre * rre - him * rim
    o_ref[:, _DH:] = hre * rim + him * rre


_tc_rot = pl.pallas_call(
    _tc_rot_body,
    grid=(R, N // _BLK),
    in_specs=[
        pl.BlockSpec((_BLK, D), lambda t, j: (j, 0)),
        pl.BlockSpec((R, D), lambda t, j: (0, 0)),
    ],
    out_specs=pl.BlockSpec((_BLK, D), lambda t, j: (t * (N // _BLK) + j, 0)),
    out_shape=jax.ShapeDtypeStruct((R * N, D), jnp.float32),
)


def _tc_conv_body(p0_ref, p1_ref, w_ref, b_ref, o_ref):
    acc = p0_ref[...] + p1_ref[...]
    o_ref[...] = jnp.maximum(acc @ w_ref[...] + b_ref[...], 0.0)


_tc_conv = pl.pallas_call(
    _tc_conv_body,
    grid=(N // _BLK,),
    in_specs=[
        pl.BlockSpec((_BLK, D), lambda i: (i, 0)),
        pl.BlockSpec((_BLK, D), lambda i: (i, 0)),
        pl.BlockSpec((D, D), lambda i: (0, 0)),
        pl.BlockSpec((1, D), lambda i: (0, 0)),
    ],
    out_specs=pl.BlockSpec((_BLK, D), lambda i: (i, 0)),
    out_shape=jax.ShapeDtypeStruct((N, D), jnp.float32),
)


def _tc_head_body(p0_ref, p1_ref, w2_ref, b2_ref, wm1_ref, bm1_ref,
                  wm2_ref, bm2_ref, wav_ref, bav_ref, o_ref):
    x = jnp.maximum((p0_ref[...] + p1_ref[...]) @ w2_ref[...] + b2_ref[...], 0.0)
    h = jnp.maximum(x @ wm1_ref[...] + bm1_ref[...], 0.0)
    h = jnp.maximum(h @ wm2_ref[...] + bm2_ref[...], 0.0)
    av = h @ wav_ref[...] + bav_ref[...]
    adv = av[:, :A]
    val = av[:, A:A + 1]
    o_ref[...] = val + adv - jnp.mean(adv, axis=-1, keepdims=True)


_H = 128

_tc_head = pl.pallas_call(
    _tc_head_body,
    grid=(N // _BLK,),
    in_specs=[
        pl.BlockSpec((_BLK, D), lambda i: (i, 0)),
        pl.BlockSpec((_BLK, D), lambda i: (i, 0)),
        pl.BlockSpec((D, D), lambda i: (0, 0)),
        pl.BlockSpec((1, D), lambda i: (0, 0)),
        pl.BlockSpec((D, _H), lambda i: (0, 0)),
        pl.BlockSpec((1, _H), lambda i: (0, 0)),
        pl.BlockSpec((_H, _H), lambda i: (0, 0)),
        pl.BlockSpec((1, _H), lambda i: (0, 0)),
        pl.BlockSpec((_H, A + 1), lambda i: (0, 0)),
        pl.BlockSpec((1, A + 1), lambda i: (0, 0)),
    ],
    out_specs=pl.BlockSpec((_BLK, A), lambda i: (i, 0)),
    out_shape=jax.ShapeDtypeStruct((N, A), jnp.float32),
)


def kernel(edge_index, edge_type, entity_emb, rel_emb, W1, b1, W2, b2,
           Wm1, bm1, Wm2, bm2, Wa, ba, Wv, bv):
    src = edge_index[0].astype(jnp.int32)
    dst = edge_index[1].astype(jnp.int32)
    typ = edge_type.astype(jnp.int32)
    cidx = (typ * N + src).reshape(NW, EPW)
    dst3 = dst.reshape(NW, NCHUNK, CH)

    t1 = _tc_rot(entity_emb, rel_emb)
    p = _sc_conv(cidx, dst3, t1)
    x1 = _tc_conv(p[:N], p[N:], W1, b1.reshape(1, D))
    t2 = _tc_rot(x1, rel_emb)
    p2 = _sc_conv(cidx, dst3, t2)

    wav = jnp.concatenate([Wa, Wv], axis=1)
    bav = jnp.concatenate([ba, bv]).reshape(1, A + 1)
    return _tc_head(p2[:N], p2[N:], W2, b2.reshape(1, D),
                    Wm1, bm1.reshape(1, _H), Wm2, bm2.reshape(1, _H),
                    wav, bav)


# speedup vs baseline: 3.9674x; 1.2609x over previous
import functools

import jax
import jax.numpy as jnp
from jax import lax
from jax.experimental import pallas as pl
from jax.experimental.pallas import tpu as pltpu
from jax.experimental.pallas import tpu_sc as plsc

N = 10000
E = 320000
D = 128
R = 32
A = 5

NC = 2
NS = 16
NW = NC * NS
EPW = E // NW
CH = 80
NCHUNK = EPW // CH
RPT = 624

_mesh = plsc.VectorSubcoreMesh(core_axis_name="c", subcore_axis_name="s")


@functools.partial(
    pl.kernel,
    out_type=jax.ShapeDtypeStruct((NC * N, D), jnp.float32),
    mesh=_mesh,
    scratch_types=[
        pltpu.VMEM((EPW,), jnp.int32),
        pltpu.VMEM((NCHUNK, CH), jnp.int32),
        [pltpu.VMEM((CH, D), jnp.float32)] * 2,
        pltpu.MemorySpace.VMEM_SHARED((N, D), jnp.float32),
        [pltpu.SemaphoreType.DMA] * 2,
    ],
)
def _sc_conv(cidx_hbm, dst_hbm, t_hbm, out_hbm,
             cidx_v, dst_v, rows, agg, semx):
    c = lax.axis_index("c")
    s = lax.axis_index("s")
    wid = c * NS + s

    pltpu.sync_copy(cidx_hbm.at[wid], cidx_v)
    pltpu.sync_copy(dst_hbm.at[wid], dst_v)

    def _zrow(r, _):
        for j in range(D // 16):
            rows[0][r, pl.ds(j * 16, 16)] = jnp.zeros((16,), jnp.float32)
        return 0
    lax.fori_loop(0, CH, _zrow, 0)

    base = s * RPT
    for t in range(RPT // CH):
        pltpu.sync_copy(rows[0], agg.at[pl.ds(base + t * CH, CH)])
    if RPT % CH:
        pltpu.sync_copy(rows[0].at[pl.ds(0, RPT % CH)],
                        agg.at[pl.ds(base + (RPT // CH) * CH, RPT % CH)])

    @pl.when(s == NS - 1)
    def _zero_tail():
        pltpu.sync_copy(rows[0].at[pl.ds(0, 16)], agg.at[pl.ds(NS * RPT, 16)])

    plsc.subcore_barrier()

    def _fetch(k, b):
        pltpu.async_copy(t_hbm.at[cidx_v.at[pl.ds(k * CH, CH)]], rows[b], semx[b])

    def _consume(k, b):
        pltpu.make_async_copy(t_hbm.at[cidx_v.at[pl.ds(k * CH, CH)]],
                              rows[b], semx[b]).wait()
        pltpu.sync_copy(rows[b], agg.at[dst_v.at[k]], add=True)

    _fetch(0, 0)

    def _pair(i, _):
        _fetch(2 * i + 1, 1)
        _consume(2 * i, 0)
        _fetch(2 * i + 2, 0)
        _consume(2 * i + 1, 1)
        return 0

    lax.fori_loop(0, (NCHUNK - 1) // 2, _pair, 0)
    _consume(NCHUNK - 1, 0)
    plsc.subcore_barrier()

    pltpu.sync_copy(agg.at[pl.ds(base, RPT)], out_hbm.at[pl.ds(c * N + base, RPT)])

    @pl.when(s == NS - 1)
    def _write_tail():
        pltpu.sync_copy(agg.at[pl.ds(NS * RPT, 16)],
                        out_hbm.at[pl.ds(c * N + NS * RPT, 16)])


_BLK = 400
_DH = D // 2


def _rot_block(x, rel_ref, t, o_ref):
    onehot = (lax.broadcasted_iota(jnp.int32, (R, 1), 0) == t).astype(jnp.float32)
    relrow = jnp.sum(rel_ref[...] * onehot, axis=0, keepdims=True)
    hre = x[:, :_DH]
    him = x[:, _DH:]
    rre = relrow[:, :_DH]
    rim = relrow[:, _DH:]
    o_ref[:, :_DH] = hre * rre - him * rim
    o_ref[:, _DH:] = hre * rim + him * rre


def _tc_rot_body(x_ref, rel_ref, o_ref):
    _rot_block(x_ref[...], rel_ref, pl.program_id(1), o_ref)


def _tc_convrot_body(p0_ref, p1_ref, w_ref, b_ref, rel_ref, o_ref, x_s):
    @pl.when(pl.program_id(1) == 0)
    def _compute_x():
        x_s[...] = jnp.maximum(
            (p0_ref[...] + p1_ref[...]) @ w_ref[...] + b_ref[...], 0.0)

    _rot_block(x_s[...], rel_ref, pl.program_id(1), o_ref)


_tc_convrot = pl.pallas_call(
    _tc_convrot_body,
    grid=(N // _BLK, R),
    in_specs=[
        pl.BlockSpec((_BLK, D), lambda j, t: (j, 0)),
        pl.BlockSpec((_BLK, D), lambda j, t: (j, 0)),
        pl.BlockSpec((D, D), lambda j, t: (0, 0)),
        pl.BlockSpec((1, D), lambda j, t: (0, 0)),
        pl.BlockSpec((R, D), lambda j, t: (0, 0)),
    ],
    out_specs=pl.BlockSpec((_BLK, D), lambda j, t: (t * (N // _BLK) + j, 0)),
    out_shape=jax.ShapeDtypeStruct((R * N, D), jnp.float32),
    scratch_shapes=[pltpu.VMEM((_BLK, D), jnp.float32)],
)


_tc_rot = pl.pallas_call(
    _tc_rot_body,
    grid=(N // _BLK, R),
    in_specs=[
        pl.BlockSpec((_BLK, D), lambda j, t: (j, 0)),
        pl.BlockSpec((R, D), lambda j, t: (0, 0)),
    ],
    out_specs=pl.BlockSpec((_BLK, D), lambda j, t: (t * (N // _BLK) + j, 0)),
    out_shape=jax.ShapeDtypeStruct((R * N, D), jnp.float32),
)


def _tc_conv_body(p0_ref, p1_ref, w_ref, b_ref, o_ref):
    acc = p0_ref[...] + p1_ref[...]
    o_ref[...] = jnp.maximum(acc @ w_ref[...] + b_ref[...], 0.0)


_tc_conv = pl.pallas_call(
    _tc_conv_body,
    grid=(N // _BLK,),
    in_specs=[
        pl.BlockSpec((_BLK, D), lambda i: (i, 0)),
        pl.BlockSpec((_BLK, D), lambda i: (i, 0)),
        pl.BlockSpec((D, D), lambda i: (0, 0)),
        pl.BlockSpec((1, D), lambda i: (0, 0)),
    ],
    out_specs=pl.BlockSpec((_BLK, D), lambda i: (i, 0)),
    out_shape=jax.ShapeDtypeStruct((N, D), jnp.float32),
)


def _tc_head_body(p0_ref, p1_ref, w2_ref, b2_ref, wm1_ref, bm1_ref,
                  wm2_ref, bm2_ref, wav_ref, bav_ref, o_ref):
    x = jnp.maximum((p0_ref[...] + p1_ref[...]) @ w2_ref[...] + b2_ref[...], 0.0)
    h = jnp.maximum(x @ wm1_ref[...] + bm1_ref[...], 0.0)
    h = jnp.maximum(h @ wm2_ref[...] + bm2_ref[...], 0.0)
    av = h @ wav_ref[...] + bav_ref[...]
    adv = av[:, :A]
    val = av[:, A:A + 1]
    o_ref[...] = val + adv - jnp.mean(adv, axis=-1, keepdims=True)


_H = 128

_tc_head = pl.pallas_call(
    _tc_head_body,
    grid=(N // _BLK,),
    in_specs=[
        pl.BlockSpec((_BLK, D), lambda i: (i, 0)),
        pl.BlockSpec((_BLK, D), lambda i: (i, 0)),
        pl.BlockSpec((D, D), lambda i: (0, 0)),
        pl.BlockSpec((1, D), lambda i: (0, 0)),
        pl.BlockSpec((D, _H), lambda i: (0, 0)),
        pl.BlockSpec((1, _H), lambda i: (0, 0)),
        pl.BlockSpec((_H, _H), lambda i: (0, 0)),
        pl.BlockSpec((1, _H), lambda i: (0, 0)),
        pl.BlockSpec((_H, A + 1), lambda i: (0, 0)),
        pl.BlockSpec((1, A + 1), lambda i: (0, 0)),
    ],
    out_specs=pl.BlockSpec((_BLK, A), lambda i: (i, 0)),
    out_shape=jax.ShapeDtypeStruct((N, A), jnp.float32),
)


def kernel(edge_index, edge_type, entity_emb, rel_emb, W1, b1, W2, b2,
           Wm1, bm1, Wm2, bm2, Wa, ba, Wv, bv):
    src = edge_index[0].astype(jnp.int32)
    dst = edge_index[1].astype(jnp.int32)
    typ = edge_type.astype(jnp.int32)
    cidx = (typ * N + src).reshape(NW, EPW)
    dst3 = dst.reshape(NW, NCHUNK, CH)

    t1 = _tc_rot(entity_emb, rel_emb)
    p = _sc_conv(cidx, dst3, t1)
    t2 = _tc_convrot(p[:N], p[N:], W1, b1.reshape(1, D), rel_emb)
    p2 = _sc_conv(cidx, dst3, t2)

    wav = jnp.concatenate([Wa, Wv], axis=1)
    bav = jnp.concatenate([ba, bv]).reshape(1, A + 1)
    return _tc_head(p2[:N], p2[N:], W2, b2.reshape(1, D),
                    Wm1, bm1.reshape(1, _H), Wm2, bm2.reshape(1, _H),
                    wav, bav)


# speedup vs baseline: 4.0119x; 1.0112x over previous
import functools

import jax
import jax.numpy as jnp
from jax import lax
from jax.experimental import pallas as pl
from jax.experimental.pallas import tpu as pltpu
from jax.experimental.pallas import tpu_sc as plsc

N = 10000
E = 320000
D = 128
R = 32
A = 5

NC = 2
NS = 16
NW = NC * NS
EPW = E // NW
CH = 80
NCHUNK = EPW // CH
RPT = 624

_mesh = plsc.VectorSubcoreMesh(core_axis_name="c", subcore_axis_name="s")


@functools.partial(
    pl.kernel,
    out_type=jax.ShapeDtypeStruct((NC * N, D), jnp.float32),
    mesh=_mesh,
    scratch_types=[
        pltpu.VMEM((EPW,), jnp.int32),
        pltpu.VMEM((NCHUNK, CH), jnp.int32),
        [pltpu.VMEM((CH, D), jnp.float32)] * 2,
        pltpu.MemorySpace.VMEM_SHARED((N, D), jnp.float32),
        [pltpu.SemaphoreType.DMA] * 2,
    ],
)
def _sc_conv(cidx_hbm, dst_hbm, t_hbm, out_hbm,
             cidx_v, dst_v, rows, agg, semx):
    c = lax.axis_index("c")
    s = lax.axis_index("s")
    wid = c * NS + s

    pltpu.sync_copy(cidx_hbm.at[wid], cidx_v)
    pltpu.sync_copy(dst_hbm.at[wid], dst_v)

    def _zrow(r, _):
        for j in range(D // 16):
            rows[0][r, pl.ds(j * 16, 16)] = jnp.zeros((16,), jnp.float32)
        return 0
    lax.fori_loop(0, CH, _zrow, 0)

    base = s * RPT
    for t in range(RPT // CH):
        pltpu.sync_copy(rows[0], agg.at[pl.ds(base + t * CH, CH)])
    if RPT % CH:
        pltpu.sync_copy(rows[0].at[pl.ds(0, RPT % CH)],
                        agg.at[pl.ds(base + (RPT // CH) * CH, RPT % CH)])

    @pl.when(s == NS - 1)
    def _zero_tail():
        pltpu.sync_copy(rows[0].at[pl.ds(0, 16)], agg.at[pl.ds(NS * RPT, 16)])

    plsc.subcore_barrier()

    def _fetch(k, b):
        pltpu.async_copy(t_hbm.at[cidx_v.at[pl.ds(k * CH, CH)]], rows[b], semx[b])

    def _consume(k, b):
        pltpu.make_async_copy(t_hbm.at[cidx_v.at[pl.ds(k * CH, CH)]],
                              rows[b], semx[b]).wait()
        pltpu.sync_copy(rows[b], agg.at[dst_v.at[k]], add=True)

    _fetch(0, 0)

    def _pair(i, _):
        _fetch(2 * i + 1, 1)
        _consume(2 * i, 0)
        _fetch(2 * i + 2, 0)
        _consume(2 * i + 1, 1)
        return 0

    lax.fori_loop(0, (NCHUNK - 1) // 2, _pair, 0)
    _consume(NCHUNK - 1, 0)
    plsc.subcore_barrier()

    pltpu.sync_copy(agg.at[pl.ds(base, RPT)], out_hbm.at[pl.ds(c * N + base, RPT)])

    @pl.when(s == NS - 1)
    def _write_tail():
        pltpu.sync_copy(agg.at[pl.ds(NS * RPT, 16)],
                        out_hbm.at[pl.ds(c * N + NS * RPT, 16)])


_BLK = 400
_DH = D // 2


def _rot_block(x, rel_ref, t, o_ref):
    onehot = (lax.broadcasted_iota(jnp.int32, (R, 1), 0) == t).astype(jnp.float32)
    relrow = jnp.sum(rel_ref[...] * onehot, axis=0, keepdims=True)
    hre = x[:, :_DH]
    him = x[:, _DH:]
    rre = relrow[:, :_DH]
    rim = relrow[:, _DH:]
    o_ref[:, :_DH] = hre * rre - him * rim
    o_ref[:, _DH:] = hre * rim + him * rre


def _tc_rot_body(x_ref, rel_ref, o_ref):
    _rot_block(x_ref[...], rel_ref, pl.program_id(1), o_ref)


def _tc_convrot_body(p0_ref, p1_ref, w_ref, b_ref, rel_ref, o_ref, x_s):
    @pl.when(pl.program_id(1) == 0)
    def _compute_x():
        x_s[...] = jnp.maximum(
            (p0_ref[...] + p1_ref[...]) @ w_ref[...] + b_ref[...], 0.0)

    _rot_block(x_s[...], rel_ref, pl.program_id(1), o_ref)


_tc_convrot = pl.pallas_call(
    _tc_convrot_body,
    grid=(N // _BLK, R),
    in_specs=[
        pl.BlockSpec((_BLK, D), lambda j, t: (j, 0)),
        pl.BlockSpec((_BLK, D), lambda j, t: (N // _BLK + j, 0)),
        pl.BlockSpec((D, D), lambda j, t: (0, 0)),
        pl.BlockSpec((1, D), lambda j, t: (0, 0)),
        pl.BlockSpec((R, D), lambda j, t: (0, 0)),
    ],
    out_specs=pl.BlockSpec((_BLK, D), lambda j, t: (t * (N // _BLK) + j, 0)),
    out_shape=jax.ShapeDtypeStruct((R * N, D), jnp.float32),
    scratch_shapes=[pltpu.VMEM((_BLK, D), jnp.float32)],
)


_tc_rot = pl.pallas_call(
    _tc_rot_body,
    grid=(N // _BLK, R),
    in_specs=[
        pl.BlockSpec((_BLK, D), lambda j, t: (j, 0)),
        pl.BlockSpec((R, D), lambda j, t: (0, 0)),
    ],
    out_specs=pl.BlockSpec((_BLK, D), lambda j, t: (t * (N // _BLK) + j, 0)),
    out_shape=jax.ShapeDtypeStruct((R * N, D), jnp.float32),
)


def _tc_head_body(p0_ref, p1_ref, w2_ref, b2_ref, wm1_ref, bm1_ref,
                  wm2_ref, bm2_ref, wav_ref, bav_ref, o_ref):
    x = jnp.maximum((p0_ref[...] + p1_ref[...]) @ w2_ref[...] + b2_ref[...], 0.0)
    h = jnp.maximum(x @ wm1_ref[...] + bm1_ref[...], 0.0)
    h = jnp.maximum(h @ wm2_ref[...] + bm2_ref[...], 0.0)
    av = h @ wav_ref[...] + bav_ref[...]
    adv = av[:, :A]
    val = av[:, A:A + 1]
    o_ref[...] = val + adv - jnp.mean(adv, axis=-1, keepdims=True)


_H = 128

_tc_head = pl.pallas_call(
    _tc_head_body,
    grid=(N // _BLK,),
    in_specs=[
        pl.BlockSpec((_BLK, D), lambda i: (i, 0)),
        pl.BlockSpec((_BLK, D), lambda i: (N // _BLK + i, 0)),
        pl.BlockSpec((D, D), lambda i: (0, 0)),
        pl.BlockSpec((1, D), lambda i: (0, 0)),
        pl.BlockSpec((D, _H), lambda i: (0, 0)),
        pl.BlockSpec((1, _H), lambda i: (0, 0)),
        pl.BlockSpec((_H, _H), lambda i: (0, 0)),
        pl.BlockSpec((1, _H), lambda i: (0, 0)),
        pl.BlockSpec((_H, A + 1), lambda i: (0, 0)),
        pl.BlockSpec((1, A + 1), lambda i: (0, 0)),
    ],
    out_specs=pl.BlockSpec((_BLK, A), lambda i: (i, 0)),
    out_shape=jax.ShapeDtypeStruct((N, A), jnp.float32),
)


def kernel(edge_index, edge_type, entity_emb, rel_emb, W1, b1, W2, b2,
           Wm1, bm1, Wm2, bm2, Wa, ba, Wv, bv):
    src = edge_index[0].astype(jnp.int32)
    dst = edge_index[1].astype(jnp.int32)
    typ = edge_type.astype(jnp.int32)
    cidx = (typ * N + src).reshape(NW, EPW)
    dst3 = dst.reshape(NW, NCHUNK, CH)

    t1 = _tc_rot(entity_emb, rel_emb)
    p = _sc_conv(cidx, dst3, t1)
    t2 = _tc_convrot(p, p, W1, b1.reshape(1, D), rel_emb)
    p2 = _sc_conv(cidx, dst3, t2)

    wav = jnp.concatenate([Wa, Wv], axis=1)
    bav = jnp.concatenate([ba, bv]).reshape(1, A + 1)
    return _tc_head(p2, p2, W2, b2.reshape(1, D),
                    Wm1, bm1.reshape(1, _H), Wm2, bm2.reshape(1, _H),
                    wav, bav)
